# baseline (speedup 1.0000x reference)
import functools

import jax
import jax.numpy as jnp
import numpy as np
from jax.experimental import pallas as pl
from jax.experimental.pallas import tpu as pltpu

GP = 128
OUT_SIZE = 64
TCHUNK = 32


def _gru_ac_kernel(x_ref, wi_ref, wh_ref, bi_ref, bhn_ref, h0_ref,
                   w1_ref, b1_ref, w2_ref, b2_ref,
                   pol_ref, val_ref, hfin_ref,
                   xt_scr, go_scr, h_scr, Wc_scr, bc_scr, w1t_scr, w2t_scr):
    Bb, Tc, _ = x_ref.shape
    ct = pl.program_id(1)
    n_ct = pl.num_programs(1)

    @pl.when(ct == 0)
    def _init():
        Wc_scr[:GP, :3 * GP] = wi_ref[...].astype(jnp.bfloat16)
        Wc_scr[:GP, 3 * GP:] = jnp.zeros((GP, GP), jnp.bfloat16)
        Wc_scr[GP:, :2 * GP] = wh_ref[:, :2 * GP].astype(jnp.bfloat16)
        Wc_scr[GP:, 2 * GP:3 * GP] = jnp.zeros((GP, GP), jnp.bfloat16)
        Wc_scr[GP:, 3 * GP:] = wh_ref[:, 2 * GP:].astype(jnp.bfloat16)
        bc_scr[0:1, :3 * GP] = bi_ref[...]
        bc_scr[0:1, 3 * GP:] = bhn_ref[...]
        w1t_scr[...] = w1_ref[...].astype(jnp.bfloat16).T
        w2t_scr[...] = w2_ref[...].astype(jnp.bfloat16).T
        h_scr[...] = h0_ref[0]

    xt_scr[...] = jnp.transpose(x_ref[...], (1, 0, 2)).astype(jnp.bfloat16)
    Wc = Wc_scr[...]
    bias = bc_scr[...]

    hbr = Bb // 2

    def half_step(t, h, lo):
        lhs = jnp.concatenate(
            [xt_scr[t, lo:lo + hbr], h.astype(jnp.bfloat16)], axis=1)
        g = jnp.dot(lhs, Wc, preferred_element_type=jnp.float32) + bias
        rz = jax.nn.sigmoid(g[:, :2 * GP])
        r = rz[:, :GP]
        z = rz[:, GP:2 * GP]
        n = jnp.tanh(g[:, 2 * GP:3 * GP] + r * g[:, 3 * GP:])
        h_new = n + z * (h - n)
        go_scr[t, lo:lo + hbr] = h_new.astype(jnp.bfloat16)
        return h_new

    def step(t, hs):
        ha, hb = hs
        return (half_step(t, ha, 0), half_step(t, hb, hbr))

    ha, hb = jax.lax.fori_loop(
        0, Tc, step, (h_scr[:hbr], h_scr[hbr:]), unroll=8)
    h_scr[:hbr] = ha
    h_scr[hbr:] = hb

    @pl.when(ct == n_ct - 1)
    def _fin():
        hfin_ref[:hbr] = ha
        hfin_ref[hbr:] = hb

    goT = jnp.concatenate([go_scr[t].T for t in range(Tc)], axis=1)
    b1t = jnp.broadcast_to(b1_ref[...].T, (2 * GP, Tc * Bb))
    b2t = jnp.broadcast_to(b2_ref[...].T, (GP, Tc * Bb))
    h1t = jnp.maximum(
        jnp.dot(w1t_scr[...], goT, preferred_element_type=jnp.float32)
        + b1t, 0.0)
    ot = jnp.dot(w2t_scr[...], h1t.astype(jnp.bfloat16),
                 preferred_element_type=jnp.float32) + b2t
    for t in range(Tc):
        pol_ref[t] = ot[:OUT_SIZE, t * Bb:(t + 1) * Bb]
        val_ref[t] = ot[OUT_SIZE:OUT_SIZE + 1, t * Bb:(t + 1) * Bb]


@functools.partial(jax.jit, static_argnames=())
def kernel(wi, bi, wh, bhn, w1, b1, w2, b2, state, gru_hx):
    B, T, D = state.shape
    H = gru_hx.shape[-1]

    B_blk = min(B, 512)
    nb = B // B_blk
    Tc = min(TCHUNK, T)
    nT = T // Tc

    flops = (2 * T * B * 2 * GP * 4 * GP
             + 2 * T * B * GP * 2 * H
             + 2 * T * B * 2 * H * GP)
    transcendentals = T * B * 3 * GP

    def f32b(shape):
        return int(np.prod(shape)) * 4

    bytes_accessed = (f32b((T, B, D)) + f32b((B, GP))
                      + f32b((T, B, GP)) + f32b((B, GP)))

    pol_t, val_t, hfin = pl.pallas_call(
        _gru_ac_kernel,
        out_shape=(jax.ShapeDtypeStruct((T, OUT_SIZE, B), jnp.float32),
                   jax.ShapeDtypeStruct((T, 1, B), jnp.float32),
                   jax.ShapeDtypeStruct((B, GP), jnp.float32)),
        grid_spec=pltpu.PrefetchScalarGridSpec(
            num_scalar_prefetch=0,
            grid=(nb, nT),
            in_specs=[
                pl.BlockSpec((B_blk, Tc, D), lambda b, t: (b, t, 0)),
                pl.BlockSpec((D, 3 * GP), lambda b, t: (0, 0)),
                pl.BlockSpec((GP, 3 * GP), lambda b, t: (0, 0)),
                pl.BlockSpec((1, 3 * GP), lambda b, t: (0, 0)),
                pl.BlockSpec((1, GP), lambda b, t: (0, 0)),
                pl.BlockSpec((1, B_blk, GP), lambda b, t: (0, b, 0)),
                pl.BlockSpec((GP, 2 * H), lambda b, t: (0, 0)),
                pl.BlockSpec((1, 2 * H), lambda b, t: (0, 0)),
                pl.BlockSpec((2 * H, GP), lambda b, t: (0, 0)),
                pl.BlockSpec((1, GP), lambda b, t: (0, 0)),
            ],
            out_specs=(
                pl.BlockSpec((Tc, OUT_SIZE, B_blk), lambda b, t: (t, 0, b)),
                pl.BlockSpec((Tc, 1, B_blk), lambda b, t: (t, 0, b)),
                pl.BlockSpec((B_blk, GP), lambda b, t: (b, 0)),
            ),
            scratch_shapes=[pltpu.VMEM((Tc, B_blk, D), jnp.bfloat16),
                            pltpu.VMEM((Tc, B_blk, GP), jnp.bfloat16),
                            pltpu.VMEM((B_blk, GP), jnp.float32),
                            pltpu.VMEM((2 * GP, 4 * GP), jnp.bfloat16),
                            pltpu.VMEM((1, 4 * GP), jnp.float32),
                            pltpu.VMEM((2 * H, GP), jnp.bfloat16),
                            pltpu.VMEM((GP, 2 * H), jnp.bfloat16)],
        ),
        compiler_params=pltpu.CompilerParams(
            dimension_semantics=("parallel", "arbitrary"),
            vmem_limit_bytes=60 * 1024 * 1024),
        cost_estimate=pl.CostEstimate(flops=flops,
                                      transcendentals=transcendentals,
                                      bytes_accessed=bytes_accessed),
    )(state, wi, wh, bi, bhn, gru_hx, w1, b1, w2, b2)

    pol = jnp.transpose(pol_t, (2, 0, 1))
    val = jnp.transpose(val_t, (2, 0, 1))
    return pol, val, hfin[None, :, :H]


# speedup vs baseline: 1.1997x; 1.1997x over previous
import functools

import jax
import jax.numpy as jnp
import numpy as np
from jax.experimental import pallas as pl
from jax.experimental.pallas import tpu as pltpu

GP = 128
OUT_SIZE = 64
TCHUNK = 8


def _gru_ac_kernel(x_ref, wi_ref, wh_ref, bi_ref, bhn_ref, h0_ref,
                   w1_ref, b1_ref, w2_ref, b2_ref,
                   pol_ref, val_ref, hfin_ref,
                   xt_scr, go_scr, h_scr, Wc_scr, bc_scr, w1t_scr, w2t_scr):
    Bb, Tc, _ = x_ref.shape
    ct = pl.program_id(1)
    n_ct = pl.num_programs(1)

    @pl.when(ct == 0)
    def _init():
        Wc_scr[:GP, :3 * GP] = wi_ref[...].astype(jnp.bfloat16)
        Wc_scr[:GP, 3 * GP:] = jnp.zeros((GP, GP), jnp.bfloat16)
        Wc_scr[GP:, :2 * GP] = wh_ref[:, :2 * GP].astype(jnp.bfloat16)
        Wc_scr[GP:, 2 * GP:3 * GP] = jnp.zeros((GP, GP), jnp.bfloat16)
        Wc_scr[GP:, 3 * GP:] = wh_ref[:, 2 * GP:].astype(jnp.bfloat16)
        bc_scr[0:1, :3 * GP] = bi_ref[...]
        bc_scr[0:1, 3 * GP:] = bhn_ref[...]
        w1t_scr[...] = w1_ref[...].astype(jnp.bfloat16).T
        w2t_scr[...] = w2_ref[...].astype(jnp.bfloat16).T
        h_scr[...] = h0_ref[0]

    xt_scr[...] = jnp.transpose(x_ref[...], (1, 0, 2)).astype(jnp.bfloat16)
    Wc = Wc_scr[...]
    bias = bc_scr[...]

    hbr = Bb // 2

    def half_step(t, h, lo):
        lhs = jnp.concatenate(
            [xt_scr[t, lo:lo + hbr], h.astype(jnp.bfloat16)], axis=1)
        g = jnp.dot(lhs, Wc, preferred_element_type=jnp.float32) + bias
        rz = jax.nn.sigmoid(g[:, :2 * GP])
        r = rz[:, :GP]
        z = rz[:, GP:2 * GP]
        n = jnp.tanh(g[:, 2 * GP:3 * GP] + r * g[:, 3 * GP:])
        h_new = n + z * (h - n)
        go_scr[t, lo:lo + hbr] = h_new.astype(jnp.bfloat16)
        return h_new

    def step(t, hs):
        ha, hb = hs
        return (half_step(t, ha, 0), half_step(t, hb, hbr))

    ha, hb = jax.lax.fori_loop(
        0, Tc, step, (h_scr[:hbr], h_scr[hbr:]), unroll=8)
    h_scr[:hbr] = ha
    h_scr[hbr:] = hb

    @pl.when(ct == n_ct - 1)
    def _fin():
        hfin_ref[:hbr] = ha
        hfin_ref[hbr:] = hb

    goT = jnp.concatenate([go_scr[t].T for t in range(Tc)], axis=1)
    b1t = jnp.broadcast_to(b1_ref[...].T, (2 * GP, Tc * Bb))
    b2t = jnp.broadcast_to(b2_ref[...].T, (GP, Tc * Bb))
    h1t = jnp.maximum(
        jnp.dot(w1t_scr[...], goT, preferred_element_type=jnp.float32)
        + b1t, 0.0)
    ot = jnp.dot(w2t_scr[...], h1t.astype(jnp.bfloat16),
                 preferred_element_type=jnp.float32) + b2t
    for t in range(Tc):
        pol_ref[t] = ot[:OUT_SIZE, t * Bb:(t + 1) * Bb]
        val_ref[t] = ot[OUT_SIZE:OUT_SIZE + 1, t * Bb:(t + 1) * Bb]


@functools.partial(jax.jit, static_argnames=())
def kernel(wi, bi, wh, bhn, w1, b1, w2, b2, state, gru_hx):
    B, T, D = state.shape
    H = gru_hx.shape[-1]

    B_blk = min(B, 512)
    nb = B // B_blk
    Tc = min(TCHUNK, T)
    nT = T // Tc

    flops = (2 * T * B * 2 * GP * 4 * GP
             + 2 * T * B * GP * 2 * H
             + 2 * T * B * 2 * H * GP)
    transcendentals = T * B * 3 * GP

    def f32b(shape):
        return int(np.prod(shape)) * 4

    bytes_accessed = (f32b((T, B, D)) + f32b((B, GP))
                      + f32b((T, B, GP)) + f32b((B, GP)))

    pol_t, val_t, hfin = pl.pallas_call(
        _gru_ac_kernel,
        out_shape=(jax.ShapeDtypeStruct((T, OUT_SIZE, B), jnp.float32),
                   jax.ShapeDtypeStruct((T, 1, B), jnp.float32),
                   jax.ShapeDtypeStruct((B, GP), jnp.float32)),
        grid_spec=pltpu.PrefetchScalarGridSpec(
            num_scalar_prefetch=0,
            grid=(nb, nT),
            in_specs=[
                pl.BlockSpec((B_blk, Tc, D), lambda b, t: (b, t, 0)),
                pl.BlockSpec((D, 3 * GP), lambda b, t: (0, 0)),
                pl.BlockSpec((GP, 3 * GP), lambda b, t: (0, 0)),
                pl.BlockSpec((1, 3 * GP), lambda b, t: (0, 0)),
                pl.BlockSpec((1, GP), lambda b, t: (0, 0)),
                pl.BlockSpec((1, B_blk, GP), lambda b, t: (0, b, 0)),
                pl.BlockSpec((GP, 2 * H), lambda b, t: (0, 0)),
                pl.BlockSpec((1, 2 * H), lambda b, t: (0, 0)),
                pl.BlockSpec((2 * H, GP), lambda b, t: (0, 0)),
                pl.BlockSpec((1, GP), lambda b, t: (0, 0)),
            ],
            out_specs=(
                pl.BlockSpec((Tc, OUT_SIZE, B_blk), lambda b, t: (t, 0, b)),
                pl.BlockSpec((Tc, 1, B_blk), lambda b, t: (t, 0, b)),
                pl.BlockSpec((B_blk, GP), lambda b, t: (b, 0)),
            ),
            scratch_shapes=[pltpu.VMEM((Tc, B_blk, D), jnp.bfloat16),
                            pltpu.VMEM((Tc, B_blk, GP), jnp.bfloat16),
                            pltpu.VMEM((B_blk, GP), jnp.float32),
                            pltpu.VMEM((2 * GP, 4 * GP), jnp.bfloat16),
                            pltpu.VMEM((1, 4 * GP), jnp.float32),
                            pltpu.VMEM((2 * H, GP), jnp.bfloat16),
                            pltpu.VMEM((GP, 2 * H), jnp.bfloat16)],
        ),
        compiler_params=pltpu.CompilerParams(
            dimension_semantics=("parallel", "arbitrary"),
            vmem_limit_bytes=60 * 1024 * 1024),
        cost_estimate=pl.CostEstimate(flops=flops,
                                      transcendentals=transcendentals,
                                      bytes_accessed=bytes_accessed),
    )(state, wi, wh, bi, bhn, gru_hx, w1, b1, w2, b2)

    pol = jnp.transpose(pol_t, (2, 0, 1))
    val = jnp.transpose(val_t, (2, 0, 1))
    return pol, val, hfin[None, :, :H]


# speedup vs baseline: 1.2039x; 1.0035x over previous
import functools

import jax
import jax.numpy as jnp
import numpy as np
from jax.experimental import pallas as pl
from jax.experimental.pallas import tpu as pltpu

GP = 128
OUT_SIZE = 64
TCHUNK = 8


def _gru_ac_kernel(x_ref, wi_ref, wh_ref, bi_ref, bhn_ref, h0_ref,
                   w1_ref, b1_ref, w2_ref, b2_ref,
                   pol_ref, val_ref, hfin_ref,
                   xt_scr, go_scr, h_scr, Wc_scr, bc_scr, w1t_scr, w2t_scr):
    Bb, Tc, _ = x_ref.shape
    ct = pl.program_id(1)
    n_ct = pl.num_programs(1)

    @pl.when(ct == 0)
    def _init():
        wi_b = wi_ref[...]
        wh_b = wh_ref[...]
        Wc_scr[:GP, :2 * GP] = (wi_b[:, :2 * GP] * 0.5).astype(jnp.bfloat16)
        Wc_scr[:GP, 2 * GP:3 * GP] = wi_b[:, 2 * GP:].astype(jnp.bfloat16)
        Wc_scr[:GP, 3 * GP:] = jnp.zeros((GP, GP), jnp.bfloat16)
        Wc_scr[GP:, :2 * GP] = (wh_b[:, :2 * GP] * 0.5).astype(jnp.bfloat16)
        Wc_scr[GP:, 2 * GP:3 * GP] = jnp.zeros((GP, GP), jnp.bfloat16)
        Wc_scr[GP:, 3 * GP:] = wh_b[:, 2 * GP:].astype(jnp.bfloat16)
        bc_scr[0:1, :2 * GP] = bi_ref[:, :2 * GP] * 0.5
        bc_scr[0:1, 2 * GP:3 * GP] = bi_ref[:, 2 * GP:]
        bc_scr[0:1, 3 * GP:] = bhn_ref[...]
        w1t_scr[:, :GP] = w1_ref[...].astype(jnp.bfloat16).T
        w1t_scr[:, GP:GP + 1] = b1_ref[...].astype(jnp.bfloat16).T
        w1t_scr[:, GP + 1:] = jnp.zeros((2 * GP, 7), jnp.bfloat16)
        w2t_scr[...] = w2_ref[...].astype(jnp.bfloat16).T
        h_scr[...] = h0_ref[0]

    xt_scr[...] = jnp.transpose(x_ref[...], (1, 0, 2)).astype(jnp.bfloat16)
    Wc = Wc_scr[...]
    bias = bc_scr[...]

    hbr = Bb // 2

    def half_step(t, h, hb16, lo):
        lhs = jnp.concatenate([xt_scr[t, lo:lo + hbr], hb16], axis=1)
        g = jnp.dot(lhs, Wc, preferred_element_type=jnp.float32) + bias
        trz = jnp.tanh(g[:, :2 * GP])
        r = 0.5 * trz[:, :GP] + 0.5
        z = 0.5 * trz[:, GP:2 * GP] + 0.5
        n = jnp.tanh(g[:, 2 * GP:3 * GP] + r * g[:, 3 * GP:])
        h_new = n + z * (h - n)
        hb16_new = h_new.astype(jnp.bfloat16)
        go_scr[t, lo:lo + hbr] = hb16_new
        return h_new, hb16_new

    def step(t, hs):
        ha, hab, hb, hbb = hs
        ha, hab = half_step(t, ha, hab, 0)
        hb, hbb = half_step(t, hb, hbb, hbr)
        return (ha, hab, hb, hbb)

    h0a = h_scr[:hbr]
    h0b = h_scr[hbr:]
    ha, _, hb, _ = jax.lax.fori_loop(
        0, Tc, step,
        (h0a, h0a.astype(jnp.bfloat16), h0b, h0b.astype(jnp.bfloat16)),
        unroll=8)
    h_scr[:hbr] = ha
    h_scr[hbr:] = hb

    @pl.when(ct == n_ct - 1)
    def _fin():
        hfin_ref[:hbr] = ha
        hfin_ref[hbr:] = hb

    goT = jnp.concatenate(
        [jnp.concatenate([go_scr[t].T for t in range(Tc)], axis=1),
         jnp.ones((8, Tc * Bb), jnp.bfloat16)], axis=0)
    h1t = jnp.maximum(
        jnp.dot(w1t_scr[...], goT,
                preferred_element_type=jnp.float32).astype(jnp.bfloat16),
        jnp.bfloat16(0.0))
    ot = jnp.dot(w2t_scr[...], h1t,
                 preferred_element_type=jnp.float32)
    b2p = jnp.broadcast_to(b2_ref[:, :OUT_SIZE].T, (OUT_SIZE, Tc * Bb))
    b2v = jnp.broadcast_to(b2_ref[:, OUT_SIZE:OUT_SIZE + 1].T, (1, Tc * Bb))
    otp = ot[:OUT_SIZE] + b2p
    otv = ot[OUT_SIZE:OUT_SIZE + 1] + b2v
    for t in range(Tc):
        pol_ref[t] = otp[:, t * Bb:(t + 1) * Bb]
        val_ref[t] = otv[:, t * Bb:(t + 1) * Bb]


@functools.partial(jax.jit, static_argnames=())
def kernel(wi, bi, wh, bhn, w1, b1, w2, b2, state, gru_hx):
    B, T, D = state.shape
    H = gru_hx.shape[-1]

    B_blk = min(B, 512)
    nb = B // B_blk
    Tc = min(TCHUNK, T)
    nT = T // Tc

    flops = (2 * T * B * 2 * GP * 4 * GP
             + 2 * T * B * GP * 2 * H
             + 2 * T * B * 2 * H * GP)
    transcendentals = T * B * 3 * GP

    def f32b(shape):
        return int(np.prod(shape)) * 4

    bytes_accessed = (f32b((T, B, D)) + f32b((B, GP))
                      + f32b((T, B, GP)) + f32b((B, GP)))

    pol_t, val_t, hfin = pl.pallas_call(
        _gru_ac_kernel,
        out_shape=(jax.ShapeDtypeStruct((T, OUT_SIZE, B), jnp.float32),
                   jax.ShapeDtypeStruct((T, 1, B), jnp.float32),
                   jax.ShapeDtypeStruct((B, GP), jnp.float32)),
        grid_spec=pltpu.PrefetchScalarGridSpec(
            num_scalar_prefetch=0,
            grid=(nb, nT),
            in_specs=[
                pl.BlockSpec((B_blk, Tc, D), lambda b, t: (b, t, 0)),
                pl.BlockSpec((D, 3 * GP), lambda b, t: (0, 0)),
                pl.BlockSpec((GP, 3 * GP), lambda b, t: (0, 0)),
                pl.BlockSpec((1, 3 * GP), lambda b, t: (0, 0)),
                pl.BlockSpec((1, GP), lambda b, t: (0, 0)),
                pl.BlockSpec((1, B_blk, GP), lambda b, t: (0, b, 0)),
                pl.BlockSpec((GP, 2 * H), lambda b, t: (0, 0)),
                pl.BlockSpec((1, 2 * H), lambda b, t: (0, 0)),
                pl.BlockSpec((2 * H, GP), lambda b, t: (0, 0)),
                pl.BlockSpec((1, GP), lambda b, t: (0, 0)),
            ],
            out_specs=(
                pl.BlockSpec((Tc, OUT_SIZE, B_blk), lambda b, t: (t, 0, b)),
                pl.BlockSpec((Tc, 1, B_blk), lambda b, t: (t, 0, b)),
                pl.BlockSpec((B_blk, GP), lambda b, t: (b, 0)),
            ),
            scratch_shapes=[pltpu.VMEM((Tc, B_blk, D), jnp.bfloat16),
                            pltpu.VMEM((Tc, B_blk, GP), jnp.bfloat16),
                            pltpu.VMEM((B_blk, GP), jnp.float32),
                            pltpu.VMEM((2 * GP, 4 * GP), jnp.bfloat16),
                            pltpu.VMEM((1, 4 * GP), jnp.float32),
                            pltpu.VMEM((2 * H, GP + 8), jnp.bfloat16),
                            pltpu.VMEM((GP, 2 * H), jnp.bfloat16)],
        ),
        compiler_params=pltpu.CompilerParams(
            dimension_semantics=("parallel", "arbitrary"),
            vmem_limit_bytes=60 * 1024 * 1024),
        cost_estimate=pl.CostEstimate(flops=flops,
                                      transcendentals=transcendentals,
                                      bytes_accessed=bytes_accessed),
    )(state, wi, wh, bi, bhn, gru_hx, w1, b1, w2, b2)

    pol = jnp.transpose(pol_t, (2, 0, 1))
    val = jnp.transpose(val_t, (2, 0, 1))
    return pol, val, hfin[None, :, :H]
